# baseline (speedup 1.0000x reference)
import functools

import jax
import jax.numpy as jnp
from jax import lax
from jax.experimental import pallas as pl
from jax.experimental.pallas import tpu as pltpu
from jax.experimental.pallas import tpu_sc as plsc

N = 16384
C = 1000
NC = 2
NS = 16
L = 16
NW = NC * NS
NB = N // NW
W = 128
NQ = NB // W


def _sc_body(probT_hbm, tgt_hbm, rwd_hbm, out_hbm, tgt_v, rwd_v, gat_v,
             acc_v, sem):
    wid = lax.axis_index("s") * NC + lax.axis_index("c")
    base = wid * NB

    pltpu.sync_copy(tgt_hbm.at[pl.ds(base, NB)], tgt_v)
    pltpu.sync_copy(rwd_hbm.at[pl.ds(base, NB)], rwd_v)

    copies = []
    for q in range(NQ):
        cstart = pl.multiple_of(base + q * W, W)
        copies.append(pltpu.async_copy(
            probT_hbm.at[plsc.Indices(tgt_v.at[pl.ds(q * W, W)]),
                         pl.ds(cstart, W)],
            gat_v.at[pl.ds(q * W, W)], sem))

    lane = lax.broadcasted_iota(jnp.int32, (L,), 0)
    acc = jnp.zeros((L,), jnp.float32)
    for q in range(NQ):
        copies[q].wait()
        for jj in range(W // L):
            j = q * (W // L) + jj
            rows = j * L + lane
            cols = lax.bitwise_and(rows, W - 1)
            vals = plsc.load_gather(gat_v, [rows, cols])
            acc = acc + vals * rwd_v[pl.ds(j * L, L)]
    acc_v[...] = acc * (-1.0 / N)

    pltpu.sync_copy(acc_v, out_hbm.at[wid])


@jax.jit
def _ganloss(prob, targets, reward):
    mesh = plsc.VectorSubcoreMesh(core_axis_name="c", subcore_axis_name="s")
    partials = pl.kernel(
        _sc_body,
        out_type=jax.ShapeDtypeStruct((NW, L), jnp.float32),
        mesh=mesh,
        compiler_params=pltpu.CompilerParams(needs_layout_passes=False),
        scratch_types=[
            pltpu.VMEM((NB,), jnp.int32),
            pltpu.VMEM((NB,), jnp.float32),
            pltpu.VMEM((NB, W), jnp.float32),
            pltpu.VMEM((L,), jnp.float32),
            pltpu.SemaphoreType.DMA,
        ],
    )(jnp.swapaxes(prob, 0, 1), targets, reward)
    return jnp.sum(partials)


def kernel(prob, targets, reward):
    return _ganloss(prob, targets.astype(jnp.int32), reward)


# speedup vs baseline: 1.0459x; 1.0459x over previous
import functools

import jax
import jax.numpy as jnp
from jax import lax
from jax.experimental import pallas as pl
from jax.experimental.pallas import tpu as pltpu
from jax.experimental.pallas import tpu_sc as plsc

N = 16384
C = 1000
NC = 2
NS = 16
L = 16
NW = NC * NS
NB = N // NW
W = 128
NQ = NB // W


def _sc_body(probT_hbm, tgt_hbm, rwd_hbm, out_hbm, tgt_v, rwd_v, gat_v,
             acc_v, sem):
    wid = lax.axis_index("s") * NC + lax.axis_index("c")
    base = wid * NB

    pltpu.sync_copy(tgt_hbm.at[pl.ds(base, NB)], tgt_v)
    pltpu.sync_copy(rwd_hbm.at[pl.ds(base, NB)], rwd_v)

    copies = []
    for q in range(NQ):
        cstart = pl.multiple_of(base + q * W, W)
        copies.append(pltpu.async_copy(
            probT_hbm.at[plsc.Indices(tgt_v.at[pl.ds(q * W, W)]),
                         pl.ds(cstart, W)],
            gat_v.at[pl.ds(q * W, W)], sem))
    for cp in copies:
        cp.wait()

    lane = lax.broadcasted_iota(jnp.int32, (L,), 0)

    def accum(j, acc):
        rows = j * L + lane
        cols = lax.bitwise_and(rows, W - 1)
        vals = plsc.load_gather(gat_v, [rows, cols])
        return acc + vals * rwd_v[pl.ds(j * L, L)]

    acc = lax.fori_loop(0, NB // L, accum, jnp.zeros((L,), jnp.float32),
                        unroll=False)
    acc_v[...] = acc * (-1.0 / N)

    pltpu.sync_copy(acc_v, out_hbm.at[wid])


@jax.jit
def _ganloss(prob, targets, reward):
    mesh = plsc.VectorSubcoreMesh(core_axis_name="c", subcore_axis_name="s")
    partials = pl.kernel(
        _sc_body,
        out_type=jax.ShapeDtypeStruct((NW, L), jnp.float32),
        mesh=mesh,
        compiler_params=pltpu.CompilerParams(needs_layout_passes=False),
        scratch_types=[
            pltpu.VMEM((NB,), jnp.int32),
            pltpu.VMEM((NB,), jnp.float32),
            pltpu.VMEM((NB, W), jnp.float32),
            pltpu.VMEM((L,), jnp.float32),
            pltpu.SemaphoreType.DMA,
        ],
    )(jnp.swapaxes(prob, 0, 1), targets, reward)
    return jnp.sum(partials)


def kernel(prob, targets, reward):
    return _ganloss(prob, targets.astype(jnp.int32), reward)


# speedup vs baseline: 1.1328x; 1.0830x over previous
import functools

import jax
import jax.numpy as jnp
from jax import lax
from jax.experimental import pallas as pl
from jax.experimental.pallas import tpu as pltpu
from jax.experimental.pallas import tpu_sc as plsc

N = 16384
C = 1000
NC = 2
NS = 16
L = 16
NW = NC * NS
NB = N // NW
W = 128
NQ = NB // W


def _sc_body(probT_hbm, tgt_hbm, rwd_hbm, out_hbm, tgt_v, rwd_v, gat_v,
             acc_v, sem):
    wid = lax.axis_index("s") * NC + lax.axis_index("c")
    base = wid * NB

    pltpu.sync_copy(tgt_hbm.at[pl.ds(base, NB)], tgt_v)
    pltpu.sync_copy(rwd_hbm.at[pl.ds(base, NB)], rwd_v)

    def fire(q, carry):
        cstart = pl.multiple_of(base + q * W, W)
        dst0 = pl.multiple_of(q * W, W)
        pltpu.async_copy(
            probT_hbm.at[plsc.Indices(tgt_v.at[pl.ds(dst0, W)]),
                         pl.ds(cstart, W)],
            gat_v.at[pl.ds(dst0, W)], sem)
        return carry

    lax.fori_loop(0, NQ, fire, 0, unroll=False)

    def drain(q, carry):
        pltpu.make_async_copy(
            probT_hbm.at[plsc.Indices(tgt_v.at[pl.ds(0, W)]), pl.ds(0, W)],
            gat_v.at[pl.ds(0, W)], sem).wait()
        return carry

    lax.fori_loop(0, NQ, drain, 0, unroll=False)

    lane = lax.broadcasted_iota(jnp.int32, (L,), 0)

    def accum(j, acc):
        rows = j * L + lane
        cols = lax.bitwise_and(rows, W - 1)
        vals = plsc.load_gather(gat_v, [rows, cols])
        return acc + vals * rwd_v[pl.ds(j * L, L)]

    acc = lax.fori_loop(0, NB // L, accum, jnp.zeros((L,), jnp.float32),
                        unroll=False)
    acc_v[...] = acc * (-1.0 / N)

    pltpu.sync_copy(acc_v, out_hbm.at[wid])


@jax.jit
def _ganloss(prob, targets, reward):
    mesh = plsc.VectorSubcoreMesh(core_axis_name="c", subcore_axis_name="s")
    partials = pl.kernel(
        _sc_body,
        out_type=jax.ShapeDtypeStruct((NW, L), jnp.float32),
        mesh=mesh,
        compiler_params=pltpu.CompilerParams(needs_layout_passes=False),
        scratch_types=[
            pltpu.VMEM((NB,), jnp.int32),
            pltpu.VMEM((NB,), jnp.float32),
            pltpu.VMEM((NB, W), jnp.float32),
            pltpu.VMEM((L,), jnp.float32),
            pltpu.SemaphoreType.DMA,
        ],
    )(jnp.swapaxes(prob, 0, 1), targets, reward)
    return jnp.sum(partials)


def kernel(prob, targets, reward):
    return _ganloss(prob, targets.astype(jnp.int32), reward)


# speedup vs baseline: 1.1337x; 1.0008x over previous
import functools

import jax
import jax.numpy as jnp
from jax import lax
from jax.experimental import pallas as pl
from jax.experimental.pallas import tpu as pltpu
from jax.experimental.pallas import tpu_sc as plsc

N = 16384
C = 1000
NC = 2
NS = 16
L = 16
NW = NC * NS
NB = N // NW
W = 128
NQ = NB // W


def _sc_body(probT_hbm, tgt_hbm, rwd_hbm, out_hbm, tgt_v, rwd_v, gat_v,
             acc_v, sem):
    wid = lax.axis_index("s") * NC + lax.axis_index("c")
    base = wid * NB

    pltpu.sync_copy(tgt_hbm.at[pl.ds(base, NB)], tgt_v)
    pltpu.sync_copy(rwd_hbm.at[pl.ds(base, NB)], rwd_v)

    def fire(q, carry):
        cstart = pl.multiple_of(base + q * W, W)
        dst0 = pl.multiple_of(q * W, W)
        pltpu.async_copy(
            probT_hbm.at[plsc.Indices(tgt_v.at[pl.ds(dst0, W)]),
                         pl.ds(cstart, W)],
            gat_v.at[pl.ds(dst0, W)], sem)
        return carry

    lax.fori_loop(0, NQ, fire, 0, unroll=False)

    lane = lax.broadcasted_iota(jnp.int32, (L,), 0)

    def outer(q, acc):
        pltpu.make_async_copy(
            probT_hbm.at[plsc.Indices(tgt_v.at[pl.ds(0, W)]), pl.ds(0, W)],
            gat_v.at[pl.ds(0, W)], sem).wait()

        def accum(j, acc):
            rows = j * L + lane
            cols = lax.bitwise_and(rows, W - 1)
            vals = plsc.load_gather(gat_v, [rows, cols])
            return acc + vals * rwd_v[pl.ds(j * L, L)]

        return lax.fori_loop(q * (W // L), (q + 1) * (W // L), accum, acc,
                             unroll=False)

    acc = lax.fori_loop(0, NQ, outer, jnp.zeros((L,), jnp.float32),
                        unroll=False)
    acc_v[...] = acc * (-1.0 / N)

    pltpu.sync_copy(acc_v, out_hbm.at[wid])


@jax.jit
def _ganloss(prob, targets, reward):
    mesh = plsc.VectorSubcoreMesh(core_axis_name="c", subcore_axis_name="s")
    partials = pl.kernel(
        _sc_body,
        out_type=jax.ShapeDtypeStruct((NW, L), jnp.float32),
        mesh=mesh,
        compiler_params=pltpu.CompilerParams(needs_layout_passes=False),
        scratch_types=[
            pltpu.VMEM((NB,), jnp.int32),
            pltpu.VMEM((NB,), jnp.float32),
            pltpu.VMEM((NB, W), jnp.float32),
            pltpu.VMEM((L,), jnp.float32),
            pltpu.SemaphoreType.DMA,
        ],
    )(jnp.swapaxes(prob, 0, 1), targets, reward)
    return jnp.sum(partials)


def kernel(prob, targets, reward):
    return _ganloss(prob, targets.astype(jnp.int32), reward)
